# baseline (speedup 1.0000x reference)
import jax
import jax.numpy as jnp
from jax import lax
from jax.experimental import pallas as pl
from jax.experimental.pallas import tpu as pltpu

_TRIM_FRACTION = 0.25
_ANOMALY_THRESHOLD = 0.9


def _dist_kernel(x_ref, g_ref, out_ref):
    x = x_ref[...]
    g = g_ref[...]
    d = x - g
    out_ref[...] = jnp.sum(d * d, axis=1).reshape(1, 1, -1)


def _rank_lower_median(v):
    n = v.shape[0]
    col = v[:, None]
    row = v[None, :]
    ii = lax.broadcasted_iota(jnp.int32, (n, n), 0)
    jj = lax.broadcasted_iota(jnp.int32, (n, n), 1)
    m = (row < col) | ((row == col) & (jj < ii))
    rank = jnp.sum(m.astype(jnp.int32), axis=1)
    return jnp.sum(jnp.where(rank == (n - 1) // 2, v, 0.0))


def _agg_kernel(partials_ref, x_ref, out_ref, keep_ref, scal_ref):
    i = pl.program_id(0)
    n = x_ref.shape[0]

    @pl.when(i == 0)
    def _():
        dist2 = jnp.sum(partials_ref[...], axis=(0, 1))
        d = jnp.sqrt(dist2)
        med = _rank_lower_median(d)
        mad = _rank_lower_median(jnp.abs(d - med))
        thr = med + _ANOMALY_THRESHOLD * mad
        keep = (d <= thr).astype(jnp.float32)
        k = jnp.sum(keep)
        none_kept = k == 0.0
        keep_eff = jnp.where(none_kept, jnp.ones_like(keep), keep)
        k_eff = jnp.where(none_kept, jnp.float32(n), k)
        t = jnp.floor(k_eff * _TRIM_FRACTION)
        keep_ref[...] = keep_eff[:, None]
        scal_ref[0] = t
        scal_ref[1] = k_eff - t
        scal_ref[2] = k_eff - 2.0 * t

    x = x_ref[...]
    keep = keep_ref[...]
    xm = jnp.where(keep > 0.0, x, jnp.inf)
    t = scal_ref[0]
    kmt = scal_ref[1]
    count = scal_ref[2]

    rows = lax.broadcasted_iota(jnp.int32, (n, 1), 0)
    pos = rows.astype(jnp.float32)
    sel = (pos >= t) & (pos < kmt)

    def _sort_net(xs):
        k = 2
        while k <= n:
            d = k // 2
            while d >= 1:
                bit_d = (rows & d) != 0
                bit_k = (rows & k) != 0
                down = pltpu.roll(xs, d, axis=0)
                upv = pltpu.roll(xs, n - d, axis=0)
                p = jnp.where(bit_d, down, upv)
                mn = jnp.minimum(xs, p)
                mx = jnp.maximum(xs, p)
                keep_min = jnp.logical_not(jnp.logical_xor(bit_d, bit_k))
                xs = jnp.where(keep_min, mn, mx)
                d //= 2
            k *= 2
        return xs

    sub = 512
    b = x.shape[1]
    outs = []
    for c0 in range(0, b, sub):
        xs = _sort_net(xm[:, c0:c0 + sub])
        outs.append(jnp.sum(jnp.where(sel, xs, 0.0), axis=0))
    s = jnp.concatenate(outs)
    out_ref[...] = (s / count).reshape(1, 1, -1)


def kernel(updates, global_model):
    n, c = updates.shape
    block = 8192 if c % 8192 == 0 else 256
    nb = c // block
    g2d = global_model.reshape(1, c)

    partials = pl.pallas_call(
        _dist_kernel,
        grid=(nb,),
        in_specs=[
            pl.BlockSpec((n, block), lambda i: (0, i)),
            pl.BlockSpec((1, block), lambda i: (0, i)),
        ],
        out_specs=pl.BlockSpec((1, 1, n), lambda i: (i, 0, 0)),
        out_shape=jax.ShapeDtypeStruct((nb, 1, n), jnp.float32),
    )(updates, g2d)

    out3 = pl.pallas_call(
        _agg_kernel,
        grid=(nb,),
        in_specs=[
            pl.BlockSpec((nb, 1, n), lambda i: (0, 0, 0)),
            pl.BlockSpec((n, block), lambda i: (0, i)),
        ],
        out_specs=pl.BlockSpec((1, 1, block), lambda i: (i, 0, 0)),
        out_shape=jax.ShapeDtypeStruct((nb, 1, block), jnp.float32),
        scratch_shapes=[
            pltpu.VMEM((n, 1), jnp.float32),
            pltpu.SMEM((4,), jnp.float32),
        ],
    )(partials, updates)

    return out3.reshape(c)


# speedup vs baseline: 1.0485x; 1.0485x over previous
import jax
import jax.numpy as jnp
from jax import lax
from jax.experimental import pallas as pl
from jax.experimental.pallas import tpu as pltpu

_TRIM_FRACTION = 0.25
_ANOMALY_THRESHOLD = 0.9


def _dist_kernel(x_ref, g_ref, out_ref):
    x = x_ref[...]
    g = g_ref[...]
    d = x - g
    out_ref[...] = jnp.sum(d * d, axis=1).reshape(1, 1, -1)


def _rank_lower_median(v):
    n = v.shape[0]
    col = v[:, None]
    row = v[None, :]
    ii = lax.broadcasted_iota(jnp.int32, (n, n), 0)
    jj = lax.broadcasted_iota(jnp.int32, (n, n), 1)
    m = (row < col) | ((row == col) & (jj < ii))
    rank = jnp.sum(m.astype(jnp.int32), axis=1)
    return jnp.sum(jnp.where(rank == (n - 1) // 2, v, 0.0))


def _agg_kernel(partials_ref, x_ref, out_ref, keep_ref, scal_ref):
    i = pl.program_id(0)
    n = x_ref.shape[0]

    @pl.when(i == 0)
    def _():
        dist2 = jnp.sum(partials_ref[...], axis=(0, 1))
        d = jnp.sqrt(dist2)
        med = _rank_lower_median(d)
        mad = _rank_lower_median(jnp.abs(d - med))
        thr = med + _ANOMALY_THRESHOLD * mad
        keep = (d <= thr).astype(jnp.float32)
        k = jnp.sum(keep)
        none_kept = k == 0.0
        keep_eff = jnp.where(none_kept, jnp.ones_like(keep), keep)
        k_eff = jnp.where(none_kept, jnp.float32(n), k)
        t = jnp.floor(k_eff * _TRIM_FRACTION)
        keep_ref[...] = keep_eff[:, None]
        scal_ref[0] = t
        scal_ref[1] = k_eff - t
        scal_ref[2] = k_eff - 2.0 * t

    x = x_ref[...]
    keep = keep_ref[...]
    xm = jnp.where(keep > 0.0, x, jnp.inf)
    t = scal_ref[0]
    kmt = scal_ref[1]
    count = scal_ref[2]

    rows = lax.broadcasted_iota(jnp.int32, (n, 1), 0)
    pos = rows.astype(jnp.float32)
    sel = (pos >= t) & (pos < kmt)

    def _sort_net(xs):
        k = 2
        while k <= n:
            d = k // 2
            while d >= 1:
                bit_d = (rows & d) != 0
                bit_k = (rows & k) != 0
                down = pltpu.roll(xs, d, axis=0)
                upv = pltpu.roll(xs, n - d, axis=0)
                p = jnp.where(bit_d, down, upv)
                mn = jnp.minimum(xs, p)
                mx = jnp.maximum(xs, p)
                keep_min = jnp.logical_not(jnp.logical_xor(bit_d, bit_k))
                xs = jnp.where(keep_min, mn, mx)
                d //= 2
            k *= 2
        return xs

    sub = 128
    b = x.shape[1]
    outs = []
    for c0 in range(0, b, sub):
        xs = _sort_net(xm[:, c0:c0 + sub])
        outs.append(jnp.sum(jnp.where(sel, xs, 0.0), axis=0))
    s = jnp.concatenate(outs)
    out_ref[...] = (s / count).reshape(1, 1, -1)


def kernel(updates, global_model):
    n, c = updates.shape
    block = 8192 if c % 8192 == 0 else 256
    nb = c // block
    g2d = global_model.reshape(1, c)

    partials = pl.pallas_call(
        _dist_kernel,
        grid=(nb,),
        in_specs=[
            pl.BlockSpec((n, block), lambda i: (0, i)),
            pl.BlockSpec((1, block), lambda i: (0, i)),
        ],
        out_specs=pl.BlockSpec((1, 1, n), lambda i: (i, 0, 0)),
        out_shape=jax.ShapeDtypeStruct((nb, 1, n), jnp.float32),
    )(updates, g2d)

    out3 = pl.pallas_call(
        _agg_kernel,
        grid=(nb,),
        in_specs=[
            pl.BlockSpec((nb, 1, n), lambda i: (0, 0, 0)),
            pl.BlockSpec((n, block), lambda i: (0, i)),
        ],
        out_specs=pl.BlockSpec((1, 1, block), lambda i: (i, 0, 0)),
        out_shape=jax.ShapeDtypeStruct((nb, 1, block), jnp.float32),
        scratch_shapes=[
            pltpu.VMEM((n, 1), jnp.float32),
            pltpu.SMEM((4,), jnp.float32),
        ],
    )(partials, updates)

    return out3.reshape(c)


# speedup vs baseline: 1.4593x; 1.3918x over previous
import jax
import jax.numpy as jnp
from jax import lax
from jax.experimental import pallas as pl
from jax.experimental.pallas import tpu as pltpu

_TRIM_FRACTION = 0.25
_ANOMALY_THRESHOLD = 0.9


def _dist_kernel(x_ref, g_ref, out_ref):
    x = x_ref[...]
    g = g_ref[...]
    d = x - g
    out_ref[...] = jnp.sum(d * d, axis=1).reshape(1, 1, -1)


def _rank_lower_median(v):
    n = v.shape[0]
    col = v[:, None]
    row = v[None, :]
    ii = lax.broadcasted_iota(jnp.int32, (n, n), 0)
    jj = lax.broadcasted_iota(jnp.int32, (n, n), 1)
    m = (row < col) | ((row == col) & (jj < ii))
    rank = jnp.sum(m.astype(jnp.int32), axis=1)
    return jnp.sum(jnp.where(rank == (n - 1) // 2, v, 0.0))


def _agg_kernel(partials_ref, x_ref, out_ref, keep_ref, scal_ref):
    i = pl.program_id(0)
    n = x_ref.shape[0]

    @pl.when(i == 0)
    def _():
        dist2 = jnp.sum(partials_ref[...], axis=(0, 1))
        d = jnp.sqrt(dist2)
        med = _rank_lower_median(d)
        mad = _rank_lower_median(jnp.abs(d - med))
        thr = med + _ANOMALY_THRESHOLD * mad
        keep = (d <= thr).astype(jnp.float32)
        k = jnp.sum(keep)
        none_kept = k == 0.0
        keep_eff = jnp.where(none_kept, jnp.ones_like(keep), keep)
        k_eff = jnp.where(none_kept, jnp.float32(n), k)
        t = jnp.floor(k_eff * _TRIM_FRACTION)
        keep_ref[...] = keep_eff[:, None]
        scal_ref[0] = t
        scal_ref[1] = k_eff - t
        scal_ref[2] = k_eff - 2.0 * t

    x = x_ref[...]
    keep = keep_ref[...]
    xm = jnp.where(keep > 0.0, x, jnp.inf)
    t = scal_ref[0]
    kmt = scal_ref[1]
    count = scal_ref[2]

    rows = lax.broadcasted_iota(jnp.int32, (n, 1), 0)
    pos = rows.astype(jnp.float32)
    sel = (pos >= t) & (pos < kmt)

    def _cmpx(blk, d, h, asc):
        rr = lax.broadcasted_iota(jnp.int32, (h, 1), 0)
        bd = (rr & d) != 0
        down = pltpu.roll(blk, d, axis=0)
        upv = pltpu.roll(blk, h - d, axis=0)
        if asc:
            return jnp.where(bd, jnp.maximum(blk, down),
                             jnp.minimum(blk, upv))
        return jnp.where(bd, jnp.minimum(blk, down),
                         jnp.maximum(blk, upv))

    def _sort_net(xs):
        for k in (2, 4):
            d = k // 2
            while d >= 1:
                bit_d = (rows & d) != 0
                bit_k = (rows & k) != 0
                down = pltpu.roll(xs, d, axis=0)
                upv = pltpu.roll(xs, n - d, axis=0)
                p = jnp.where(bit_d, down, upv)
                mn = jnp.minimum(xs, p)
                mx = jnp.maximum(xs, p)
                keep_min = jnp.logical_not(jnp.logical_xor(bit_d, bit_k))
                xs = jnp.where(keep_min, mn, mx)
                d //= 2
        k = 8
        while k <= n:
            d = k // 2
            while d >= 1:
                if k == n:
                    xs = _cmpx(xs, d, n, True)
                else:
                    xs = jnp.concatenate(
                        [_cmpx(xs[m * k:(m + 1) * k], d, k, m % 2 == 0)
                         for m in range(n // k)], axis=0)
                d //= 2
            k *= 2
        return xs

    sub = 128
    b = x.shape[1]
    outs = []
    for c0 in range(0, b, sub):
        xs = _sort_net(xm[:, c0:c0 + sub])
        outs.append(jnp.sum(jnp.where(sel, xs, 0.0), axis=0))
    s = jnp.concatenate(outs)
    out_ref[...] = (s / count).reshape(1, 1, -1)


def kernel(updates, global_model):
    n, c = updates.shape
    block = 8192 if c % 8192 == 0 else 256
    nb = c // block
    g2d = global_model.reshape(1, c)

    partials = pl.pallas_call(
        _dist_kernel,
        grid=(nb,),
        in_specs=[
            pl.BlockSpec((n, block), lambda i: (0, i)),
            pl.BlockSpec((1, block), lambda i: (0, i)),
        ],
        out_specs=pl.BlockSpec((1, 1, n), lambda i: (i, 0, 0)),
        out_shape=jax.ShapeDtypeStruct((nb, 1, n), jnp.float32),
    )(updates, g2d)

    out3 = pl.pallas_call(
        _agg_kernel,
        grid=(nb,),
        in_specs=[
            pl.BlockSpec((nb, 1, n), lambda i: (0, 0, 0)),
            pl.BlockSpec((n, block), lambda i: (0, i)),
        ],
        out_specs=pl.BlockSpec((1, 1, block), lambda i: (i, 0, 0)),
        out_shape=jax.ShapeDtypeStruct((nb, 1, block), jnp.float32),
        scratch_shapes=[
            pltpu.VMEM((n, 1), jnp.float32),
            pltpu.SMEM((4,), jnp.float32),
        ],
    )(partials, updates)

    return out3.reshape(c)
